# baseline (speedup 1.0000x reference)
import functools

import jax
import jax.numpy as jnp
from jax import lax
from jax.experimental import pallas as pl
from jax.experimental.pallas import tpu as pltpu
from jax.experimental.pallas import tpu_sc as plsc

N = 50000
E = 800000
HID = 128
NG = 64

NCORES = 2
NSUB = 16
B = 128
TB = 6400
EP = TB * B
KSLOT = 5
DDELAY = 4
NP = 50048
NPT = NP // NSUB
CH = 4
CW = HID // CH

_P = jax.lax.Precision.DEFAULT


def _ea_body(ea_hbm, dst_hbm, zeros_hbm, out_hbm, dstv, rows, shared, semr, sems):
    c = lax.axis_index("c")
    s = lax.axis_index("s")
    nb = TB // (NCORES * NSUB)
    b0 = c * (TB // NCORES) + s * nb

    pltpu.sync_copy(zeros_hbm.at[pl.ds(s * NPT, NPT)],
                    shared.at[pl.ds(s * NPT, NPT)])
    plsc.subcore_barrier()

    pltpu.sync_copy(dst_hbm.at[pl.ds(b0, nb)], dstv)

    mceil = (nb + DDELAY + KSLOT - 1) // KSLOT

    def step(m, _):
        for k in range(KSLOT):
            i = m * KSLOT + k
            q = (k - DDELAY) % KSLOT

            @pl.when(i < nb)
            def _():
                @pl.when(i >= KSLOT)
                def _():
                    pltpu.make_async_copy(
                        rows.at[k], shared.at[dstv.at[i - KSLOT]],
                        sems.at[k]).wait()
                pltpu.async_copy(ea_hbm.at[b0 + i], rows.at[k], semr.at[k])

            j = i - DDELAY

            @pl.when(jnp.logical_and(j >= 0, j < nb))
            def _():
                pltpu.make_async_copy(ea_hbm.at[b0 + j], rows.at[q],
                                      semr.at[q]).wait()
                pltpu.async_copy(rows.at[q], shared.at[dstv.at[j]],
                                 sems.at[q], add=True)
        return 0

    lax.fori_loop(0, mceil, step, 0)

    for k in range(KSLOT):
        jj = nb - 1 - ((nb - 1 - k) % KSLOT)
        pltpu.make_async_copy(rows.at[jj % KSLOT], shared.at[dstv.at[jj]],
                              sems.at[jj % KSLOT]).wait()

    plsc.subcore_barrier()
    pltpu.sync_copy(shared.at[pl.ds(s * NPT, NPT)],
                    out_hbm.at[c, pl.ds(s * NPT, NPT)])


def _sc_ea(ea_pad3, dst2, zeros16):
    mesh = plsc.VectorSubcoreMesh(core_axis_name="c", subcore_axis_name="s")
    kb = pl.kernel(
        _ea_body,
        out_type=jax.ShapeDtypeStruct((NCORES, NP, 16), jnp.float32),
        mesh=mesh,
        compiler_params=pltpu.CompilerParams(use_tc_tiling_on_sc=False),
        scratch_types=[
            pltpu.VMEM((TB // (NCORES * NSUB), B), jnp.int32),
            pltpu.VMEM((KSLOT, B, 16), jnp.float32),
            pltpu.VMEM_SHARED((NP, 16), jnp.float32),
            pltpu.SemaphoreType.DMA((KSLOT,)),
            pltpu.SemaphoreType.DMA((KSLOT,)),
        ],
    )
    return kb(ea_pad3, dst2, zeros16)


SB = 10
NSUP = (TB // NSUB) // SB


def _agg_body(h_hbm, src_hbm, dst_hbm, zeros_hbm, out_hbm,
              srcv, dstv, rows, shared, semr, sems, semi):
    c = lax.axis_index("c")
    s = lax.axis_index("s")
    nb = TB // NSUB
    b0 = s * nb

    def idx_issue(msup):
        sl = lax.rem(msup, 2)
        pltpu.async_copy(src_hbm.at[pl.ds(b0 + msup * SB, SB)],
                         srcv.at[sl], semi.at[sl])
        pltpu.async_copy(dst_hbm.at[pl.ds(b0 + msup * SB, SB)],
                         dstv.at[sl], semi.at[sl])

    def idx_wait(msup):
        sl = lax.rem(msup, 2)
        pltpu.make_async_copy(src_hbm.at[pl.ds(b0 + msup * SB, SB)],
                              srcv.at[sl], semi.at[sl]).wait()
        pltpu.make_async_copy(dst_hbm.at[pl.ds(b0 + msup * SB, SB)],
                              dstv.at[sl], semi.at[sl]).wait()

    def srow(msup, k2):
        return srcv.at[lax.rem(msup + (k2 // SB), 2), k2 % SB]

    def drow(msup, k2):
        return dstv.at[lax.rem(msup + (k2 // SB), 2), k2 % SB]

    for cc in range(CH // NCORES):
        g = c * (CH // NCORES) + cc

        pltpu.sync_copy(zeros_hbm.at[pl.ds(s * NPT, NPT)],
                        shared.at[pl.ds(s * NPT, NPT)])
        plsc.subcore_barrier()
        idx_issue(0)

        def step(msup, _):
            idx_wait(msup)
            for k2 in range(SB):
                i = msup * SB + k2
                k = k2 % KSLOT
                q = (k2 - DDELAY) % KSLOT

                @pl.when(i >= KSLOT)
                def _():
                    pltpu.make_async_copy(
                        rows.at[k], shared.at[drow(msup, k2 - KSLOT)],
                        sems.at[k]).wait()

                pltpu.async_copy(h_hbm.at[g].at[srow(msup, k2)],
                                 rows.at[k], semr.at[k])

                j = i - DDELAY

                @pl.when(j >= 0)
                def _():
                    pltpu.make_async_copy(
                        h_hbm.at[g].at[srow(msup, k2 - DDELAY)],
                        rows.at[q], semr.at[q]).wait()
                    pltpu.async_copy(rows.at[q],
                                     shared.at[drow(msup, k2 - DDELAY)],
                                     sems.at[q], add=True)

                if k2 == KSLOT:
                    @pl.when(msup + 1 < NSUP)
                    def _():
                        idx_issue(msup + 1)
            return 0

        lax.fori_loop(0, NSUP, step, 0)

        last = NSUP - 1
        for k2 in range(SB, SB + DDELAY):
            q = (k2 - DDELAY) % KSLOT
            jx = last * SB + k2 - DDELAY
            pltpu.make_async_copy(h_hbm.at[g].at[srow(last, k2 - DDELAY)],
                                  rows.at[q], semr.at[q]).wait()
            pltpu.async_copy(rows.at[q], shared.at[drow(last, k2 - DDELAY)],
                             sems.at[q], add=True)
        for k2 in range(SB - KSLOT, SB):
            k = k2 % KSLOT
            jx = last * SB + k2
            pltpu.make_async_copy(rows.at[k], shared.at[drow(last, k2)],
                                  sems.at[k]).wait()

        plsc.subcore_barrier()
        pltpu.sync_copy(shared.at[pl.ds(s * NPT, NPT)],
                        out_hbm.at[g, pl.ds(s * NPT, NPT)])
        plsc.subcore_barrier()


def _sc_agg(h4, src2, dst2, zeros32):
    mesh = plsc.VectorSubcoreMesh(core_axis_name="c", subcore_axis_name="s")
    kb = pl.kernel(
        _agg_body,
        out_type=jax.ShapeDtypeStruct((CH, NP, CW), jnp.float32),
        mesh=mesh,
        compiler_params=pltpu.CompilerParams(use_tc_tiling_on_sc=False),
        scratch_types=[
            pltpu.VMEM((2, SB, B), jnp.int32),
            pltpu.VMEM((2, SB, B), jnp.int32),
            pltpu.VMEM((KSLOT, B, CW), jnp.float32),
            pltpu.VMEM_SHARED((NP, CW), jnp.float32),
            pltpu.SemaphoreType.DMA((KSLOT,)),
            pltpu.SemaphoreType.DMA((KSLOT,)),
            pltpu.SemaphoreType.DMA((2,)),
        ],
    )
    return kb(h4, src2, dst2, zeros32)


R = 2000
G = N // R


def _k0_body(x_ref, w_ref, b_ref, eap_ref, h4_ref, eaf_ref):
    h = jnp.dot(x_ref[...], w_ref[...], precision=_P,
                preferred_element_type=jnp.float32) + b_ref[...]
    for gg in range(CH):
        h4_ref[gg] = h[:, gg * CW:(gg + 1) * CW]
    eaf_ref[...] = eap_ref[0] + eap_ref[1]


def _tc_in(x, w, b, eap):
    return pl.pallas_call(
        _k0_body,
        grid=(G,),
        in_specs=[
            pl.BlockSpec((R, x.shape[1]), lambda i: (i, 0)),
            pl.BlockSpec(w.shape, lambda i: (0, 0)),
            pl.BlockSpec((1, HID), lambda i: (0, 0)),
            pl.BlockSpec((2, R, 16), lambda i: (0, i, 0)),
        ],
        out_specs=[
            pl.BlockSpec((CH, R, CW), lambda i: (0, i, 0)),
            pl.BlockSpec((R, 16), lambda i: (i, 0)),
        ],
        out_shape=[
            jax.ShapeDtypeStruct((CH, N, CW), jnp.float32),
            jax.ShapeDtypeStruct((N, 16), jnp.float32),
        ],
    )(x, w, b, eap)


def _k1_body(h_ref, a4_ref, ea_ref, wx_ref, w1_ref, b1_ref, eps_ref,
             u_ref, st_ref):
    hcat = jnp.concatenate([h_ref[gg] for gg in range(CH)], axis=-1)
    agg = jnp.concatenate([a4_ref[gg] for gg in range(CH)], axis=-1)
    pre = ((1.0 + eps_ref[0, 0]) * hcat + agg
           + jnp.dot(ea_ref[...], wx_ref[...], precision=_P,
                     preferred_element_type=jnp.float32))
    u = jnp.dot(pre, w1_ref[...], precision=_P,
                preferred_element_type=jnp.float32) + b1_ref[...]
    u_ref[...] = u

    @pl.when(pl.program_id(0) == 0)
    def _():
        st_ref[...] = jnp.zeros_like(st_ref)

    rid = lax.broadcasted_iota(jnp.int32, st_ref.shape, 0)
    st_ref[...] += jnp.where(rid == 0, jnp.sum(u, axis=0, keepdims=True),
                             jnp.where(rid == 1,
                                       jnp.sum(u * u, axis=0, keepdims=True),
                                       0.0))


def _tc_k1(h4, agg4, eaf, wx, w1, b1, eps):
    d2 = w1.shape[1]
    return pl.pallas_call(
        _k1_body,
        grid=(G,),
        in_specs=[
            pl.BlockSpec((CH, R, CW), lambda i: (0, i, 0)),
            pl.BlockSpec((CH, R, CW), lambda i: (0, i, 0)),
            pl.BlockSpec((R, 16), lambda i: (i, 0)),
            pl.BlockSpec((16, HID), lambda i: (0, 0)),
            pl.BlockSpec((HID, d2), lambda i: (0, 0)),
            pl.BlockSpec((1, d2), lambda i: (0, 0)),
            pl.BlockSpec((1, 1), lambda i: (0, 0)),
        ],
        out_specs=[
            pl.BlockSpec((R, d2), lambda i: (i, 0)),
            pl.BlockSpec((8, d2), lambda i: (0, 0)),
        ],
        out_shape=[
            jax.ShapeDtypeStruct((N, d2), jnp.float32),
            jax.ShapeDtypeStruct((8, d2), jnp.float32),
        ],
    )(h4, agg4, eaf, wx, w1, b1, eps)


def _k2_body(u_ref, st_ref, g_ref, bt_ref, w_ref, b_ref, v_ref, st2_ref):
    m = st_ref[0:1, :] / N
    var = jnp.maximum(st_ref[1:2, :] / N - m * m, 0.0)
    a = g_ref[...] * lax.rsqrt(var + 1e-5)
    cb = bt_ref[...] - m * a
    un = jnp.maximum(u_ref[...] * a + cb, 0.0)
    v = jnp.dot(un, w_ref[...], precision=_P,
                preferred_element_type=jnp.float32) + b_ref[...]
    v_ref[...] = v

    @pl.when(pl.program_id(0) == 0)
    def _():
        st2_ref[...] = jnp.zeros_like(st2_ref)

    rid = lax.broadcasted_iota(jnp.int32, st2_ref.shape, 0)
    st2_ref[...] += jnp.where(rid == 0, jnp.sum(v, axis=0, keepdims=True),
                              jnp.where(rid == 1,
                                        jnp.sum(v * v, axis=0, keepdims=True),
                                        0.0))


def _tc_k2(u, st1, g1, bt1, w2, b2):
    d1 = u.shape[1]
    d2 = w2.shape[1]
    return pl.pallas_call(
        _k2_body,
        grid=(G,),
        in_specs=[
            pl.BlockSpec((R, d1), lambda i: (i, 0)),
            pl.BlockSpec((8, d1), lambda i: (0, 0)),
            pl.BlockSpec((1, d1), lambda i: (0, 0)),
            pl.BlockSpec((1, d1), lambda i: (0, 0)),
            pl.BlockSpec((d1, d2), lambda i: (0, 0)),
            pl.BlockSpec((1, d2), lambda i: (0, 0)),
        ],
        out_specs=[
            pl.BlockSpec((R, d2), lambda i: (i, 0)),
            pl.BlockSpec((8, d2), lambda i: (0, 0)),
        ],
        out_shape=[
            jax.ShapeDtypeStruct((N, d2), jnp.float32),
            jax.ShapeDtypeStruct((8, d2), jnp.float32),
        ],
    )(u, st1, g1, bt1, w2, b2)


def _k3_body(v_ref, st_ref, g_ref, bt_ref, h_ref, lg_ref, lb_ref, h4_out):
    m = st_ref[0:1, :] / N
    var = jnp.maximum(st_ref[1:2, :] / N - m * m, 0.0)
    a = g_ref[...] * lax.rsqrt(var + 1e-5)
    cb = bt_ref[...] - m * a
    hcat = jnp.concatenate([h_ref[gg] for gg in range(CH)], axis=-1)
    t = jnp.maximum(v_ref[...] * a + cb, 0.0) + hcat
    lm = jnp.mean(t, axis=-1, keepdims=True)
    lv = jnp.mean((t - lm) * (t - lm), axis=-1, keepdims=True)
    hn = (t - lm) * lax.rsqrt(lv + 1e-5) * lg_ref[...] + lb_ref[...]
    for gg in range(CH):
        h4_out[gg] = hn[:, gg * CW:(gg + 1) * CW]


def _tc_k3(v, st2, g2, bt2, h4, lg, lb):
    return pl.pallas_call(
        _k3_body,
        grid=(G,),
        in_specs=[
            pl.BlockSpec((R, HID), lambda i: (i, 0)),
            pl.BlockSpec((8, HID), lambda i: (0, 0)),
            pl.BlockSpec((1, HID), lambda i: (0, 0)),
            pl.BlockSpec((1, HID), lambda i: (0, 0)),
            pl.BlockSpec((CH, R, CW), lambda i: (0, i, 0)),
            pl.BlockSpec((1, HID), lambda i: (0, 0)),
            pl.BlockSpec((1, HID), lambda i: (0, 0)),
        ],
        out_specs=pl.BlockSpec((CH, R, CW), lambda i: (0, i, 0)),
        out_shape=jax.ShapeDtypeStruct((CH, N, CW), jnp.float32),
    )(v, st2, g2, bt2, h4, lg, lb)


def _k4_body(h0_ref, h1_ref, h2_ref, h3_ref, w10, w11, w12, w13, b1_ref,
             w2_ref, b2_ref, batch_ref, ne_ref, ge_ref, psum, pcnt):
    hr = [h0_ref, h1_ref, h2_ref, h3_ref]
    wr = [w10, w11, w12, w13]
    acc = b1_ref[...]
    for ii in range(4):
        hcat = jnp.concatenate([hr[ii][gg] for gg in range(CH)], axis=-1)
        acc += jnp.dot(hcat, wr[ii][...], precision=_P,
                       preferred_element_type=jnp.float32)
    acc = jnp.maximum(acc, 0.0)
    ne = jnp.dot(acc, w2_ref[...], precision=_P,
                 preferred_element_type=jnp.float32) + b2_ref[...]
    ne_ref[...] = ne

    @pl.when(pl.program_id(0) == 0)
    def _():
        psum[...] = jnp.zeros_like(psum)
        pcnt[...] = jnp.zeros_like(pcnt)

    bt = batch_ref[0, 0, :]
    onehot = (bt[:, None] == lax.broadcasted_iota(jnp.int32, (R, NG), 1))
    onehot = onehot.astype(jnp.float32)
    dn = (((0,), (0,)), ((), ()))
    psum[...] += lax.dot_general(onehot, ne, dn, precision=_P,
                                 preferred_element_type=jnp.float32)
    pcnt[...] += lax.dot_general(onehot, jnp.ones((R, HID), jnp.float32), dn,
                                 precision=_P,
                                 preferred_element_type=jnp.float32)

    @pl.when(pl.program_id(0) == G - 1)
    def _():
        ge_ref[...] = psum[...] / jnp.maximum(pcnt[...], 1.0)


def _tc_k4(hs, w1s, b1, w2, b2, batch3):
    return pl.pallas_call(
        _k4_body,
        grid=(G,),
        in_specs=(
            [pl.BlockSpec((CH, R, CW), lambda i: (0, i, 0)) for _ in range(4)]
            + [pl.BlockSpec((HID, HID), lambda i: (0, 0)) for _ in range(4)]
            + [
                pl.BlockSpec((1, HID), lambda i: (0, 0)),
                pl.BlockSpec((HID, HID), lambda i: (0, 0)),
                pl.BlockSpec((1, HID), lambda i: (0, 0)),
                pl.BlockSpec((1, 1, R), lambda i: (i, 0, 0)),
            ]
        ),
        out_specs=[
            pl.BlockSpec((R, HID), lambda i: (i, 0)),
            pl.BlockSpec((NG, HID), lambda i: (0, 0)),
        ],
        out_shape=[
            jax.ShapeDtypeStruct((N, HID), jnp.float32),
            jax.ShapeDtypeStruct((NG, HID), jnp.float32),
        ],
        scratch_shapes=[
            pltpu.VMEM((NG, HID), jnp.float32),
            pltpu.VMEM((NG, HID), jnp.float32),
        ],
    )(*hs, *w1s, b1, w2, b2, batch3)


def kernel(x, edge_attr, edge_index, batch, params):
    pad = EP - E
    src2 = jnp.concatenate(
        [edge_index[0], jnp.zeros((pad,), jnp.int32)]).reshape(TB, B)
    dst2 = jnp.concatenate(
        [edge_index[1], jnp.full((pad,), N, jnp.int32)]).reshape(TB, B)
    ea_pad3 = jnp.concatenate(
        [jnp.concatenate([edge_attr, jnp.ones((E, 1), jnp.float32),
                          jnp.zeros((E, 12), jnp.float32)], axis=1),
         jnp.zeros((pad, 16), jnp.float32)], axis=0).reshape(TB, B, 16)
    zeros16 = jnp.zeros((NP, 16), jnp.float32)
    zeros32 = jnp.zeros((NP, CW), jnp.float32)
    batch3 = batch.reshape(G, 1, R)

    eap = _sc_ea(ea_pad3, dst2, zeros16)

    h4, eaf = _tc_in(x, params['in_W'], params['in_b'].reshape(1, HID), eap)
    hs = [h4]
    for lp in params['layers']:
        wx = jnp.concatenate(
            [lp['We'], lp['be'].reshape(1, HID),
             jnp.zeros((12, HID), jnp.float32)], axis=0)
        agg4 = _sc_agg(h4, src2, dst2, zeros32)
        u, st1 = _tc_k1(h4, agg4, eaf, wx, lp['W1'],
                        lp['b1'].reshape(1, -1), lp['eps'].reshape(1, 1))
        v, st2 = _tc_k2(u, st1, lp['g1'].reshape(1, -1),
                        lp['bt1'].reshape(1, -1), lp['W2'],
                        lp['b2'].reshape(1, -1))
        h4 = _tc_k3(v, st2, lp['g2'].reshape(1, -1),
                    lp['bt2'].reshape(1, -1), h4,
                    lp['ln_g'].reshape(1, -1), lp['ln_b'].reshape(1, -1))
        hs.append(h4)

    w1s = [params['out_W1'][i * HID:(i + 1) * HID] for i in range(4)]
    node_emb, graph_emb = _tc_k4(
        hs, w1s, params['out_b1'].reshape(1, HID), params['out_W2'],
        params['out_b2'].reshape(1, HID), batch3)
    return node_emb, graph_emb


# speedup vs baseline: 1.0542x; 1.0542x over previous
import functools

import jax
import jax.numpy as jnp
from jax import lax
from jax.experimental import pallas as pl
from jax.experimental.pallas import tpu as pltpu
from jax.experimental.pallas import tpu_sc as plsc

N = 50000
E = 800000
HID = 128
NG = 64

NCORES = 2
NSUB = 16
B = 128
TB = 6400
EP = TB * B
KSLOT = 5
DDELAY = 4
NP = 50048
NPT = NP // NSUB
CH = 4
CW = HID // CH

_P = jax.lax.Precision.DEFAULT


def _ea_body(ea_hbm, dst_hbm, zeros_hbm, out_hbm, dstv, rows, shared, semr, sems):
    c = lax.axis_index("c")
    s = lax.axis_index("s")
    nb = TB // (NCORES * NSUB)
    b0 = c * (TB // NCORES) + s * nb

    pltpu.sync_copy(zeros_hbm.at[pl.ds(s * NPT, NPT)],
                    shared.at[pl.ds(s * NPT, NPT)])
    plsc.subcore_barrier()

    pltpu.sync_copy(dst_hbm.at[pl.ds(b0, nb)], dstv)

    mceil = (nb + DDELAY + KSLOT - 1) // KSLOT

    def step(m, _):
        for k in range(KSLOT):
            i = m * KSLOT + k
            q = (k - DDELAY) % KSLOT

            @pl.when(i < nb)
            def _():
                @pl.when(i >= KSLOT)
                def _():
                    pltpu.make_async_copy(
                        rows.at[k], shared.at[dstv.at[i - KSLOT]],
                        sems.at[k]).wait()
                pltpu.async_copy(ea_hbm.at[b0 + i], rows.at[k], semr.at[k])

            j = i - DDELAY

            @pl.when(jnp.logical_and(j >= 0, j < nb))
            def _():
                pltpu.make_async_copy(ea_hbm.at[b0 + j], rows.at[q],
                                      semr.at[q]).wait()
                pltpu.async_copy(rows.at[q], shared.at[dstv.at[j]],
                                 sems.at[q], add=True)
        return 0

    lax.fori_loop(0, mceil, step, 0)

    for k in range(KSLOT):
        jj = nb - 1 - ((nb - 1 - k) % KSLOT)
        pltpu.make_async_copy(rows.at[jj % KSLOT], shared.at[dstv.at[jj]],
                              sems.at[jj % KSLOT]).wait()

    plsc.subcore_barrier()
    pltpu.sync_copy(shared.at[pl.ds(s * NPT, NPT)],
                    out_hbm.at[c, pl.ds(s * NPT, NPT)])


def _sc_ea(ea_pad3, dst2, zeros16):
    mesh = plsc.VectorSubcoreMesh(core_axis_name="c", subcore_axis_name="s")
    kb = pl.kernel(
        _ea_body,
        out_type=jax.ShapeDtypeStruct((NCORES, NP, 16), jnp.float32),
        mesh=mesh,
        compiler_params=pltpu.CompilerParams(use_tc_tiling_on_sc=False),
        scratch_types=[
            pltpu.VMEM((TB // (NCORES * NSUB), B), jnp.int32),
            pltpu.VMEM((KSLOT, B, 16), jnp.float32),
            pltpu.VMEM_SHARED((NP, 16), jnp.float32),
            pltpu.SemaphoreType.DMA((KSLOT,)),
            pltpu.SemaphoreType.DMA((KSLOT,)),
        ],
    )
    return kb(ea_pad3, dst2, zeros16)


SB = 10
NSUP = (TB // NSUB) // SB


def _agg_body(h_hbm, src_hbm, dst_hbm, zeros_hbm, out_hbm,
              srcv, dstv, rows, shared, semr, sems, semi):
    c = lax.axis_index("c")
    s = lax.axis_index("s")
    nb = TB // NSUB
    b0 = s * nb

    def idx_issue(msup):
        sl = lax.rem(msup, 2)
        pltpu.async_copy(src_hbm.at[pl.ds(b0 + msup * SB, SB)],
                         srcv.at[sl], semi.at[sl])
        pltpu.async_copy(dst_hbm.at[pl.ds(b0 + msup * SB, SB)],
                         dstv.at[sl], semi.at[sl])

    def idx_wait(msup):
        sl = lax.rem(msup, 2)
        pltpu.make_async_copy(src_hbm.at[pl.ds(b0 + msup * SB, SB)],
                              srcv.at[sl], semi.at[sl]).wait()
        pltpu.make_async_copy(dst_hbm.at[pl.ds(b0 + msup * SB, SB)],
                              dstv.at[sl], semi.at[sl]).wait()

    def srow(msup, k2):
        return srcv.at[lax.rem(msup + (k2 // SB), 2), k2 % SB]

    def drow(msup, k2):
        return dstv.at[lax.rem(msup + (k2 // SB), 2), k2 % SB]

    for cc in range(CH // NCORES):
        g = c * (CH // NCORES) + cc

        pltpu.sync_copy(zeros_hbm.at[pl.ds(s * NPT, NPT)],
                        shared.at[pl.ds(s * NPT, NPT)])
        plsc.subcore_barrier()
        idx_issue(0)

        def step(msup, _):
            idx_wait(msup)
            for k2 in range(SB):
                i = msup * SB + k2
                k = k2 % KSLOT
                q = (k2 - DDELAY) % KSLOT

                @pl.when(i >= KSLOT)
                def _():
                    pltpu.make_async_copy(
                        rows.at[k], shared.at[drow(msup, k2 - KSLOT)],
                        sems.at[k]).wait()

                pltpu.async_copy(h_hbm.at[g].at[srow(msup, k2)],
                                 rows.at[k], semr.at[k])

                j = i - DDELAY

                @pl.when(j >= 0)
                def _():
                    pltpu.make_async_copy(
                        h_hbm.at[g].at[srow(msup, k2 - DDELAY)],
                        rows.at[q], semr.at[q]).wait()
                    pltpu.async_copy(rows.at[q],
                                     shared.at[drow(msup, k2 - DDELAY)],
                                     sems.at[q], add=True)

                if k2 == KSLOT:
                    @pl.when(msup + 1 < NSUP)
                    def _():
                        idx_issue(msup + 1)
            return 0

        lax.fori_loop(0, NSUP, step, 0)

        last = NSUP - 1
        for k2 in range(SB, SB + DDELAY):
            q = (k2 - DDELAY) % KSLOT
            jx = last * SB + k2 - DDELAY
            pltpu.make_async_copy(h_hbm.at[g].at[srow(last, k2 - DDELAY)],
                                  rows.at[q], semr.at[q]).wait()
            pltpu.async_copy(rows.at[q], shared.at[drow(last, k2 - DDELAY)],
                             sems.at[q], add=True)
        for k2 in range(SB - KSLOT, SB):
            k = k2 % KSLOT
            jx = last * SB + k2
            pltpu.make_async_copy(rows.at[k], shared.at[drow(last, k2)],
                                  sems.at[k]).wait()

        plsc.subcore_barrier()
        pltpu.sync_copy(shared.at[pl.ds(s * NPT, NPT)],
                        out_hbm.at[g, pl.ds(s * NPT, NPT)])
        plsc.subcore_barrier()


def _sc_agg(h4, src2, dst2, zeros32):
    mesh = plsc.VectorSubcoreMesh(core_axis_name="c", subcore_axis_name="s")
    kb = pl.kernel(
        _agg_body,
        out_type=jax.ShapeDtypeStruct((CH, NP, CW), jnp.float32),
        mesh=mesh,
        compiler_params=pltpu.CompilerParams(use_tc_tiling_on_sc=False),
        scratch_types=[
            pltpu.VMEM((2, SB, B), jnp.int32),
            pltpu.VMEM((2, SB, B), jnp.int32),
            pltpu.VMEM((KSLOT, B, CW), jnp.float32),
            pltpu.VMEM_SHARED((NP, CW), jnp.float32),
            pltpu.SemaphoreType.DMA((KSLOT,)),
            pltpu.SemaphoreType.DMA((KSLOT,)),
            pltpu.SemaphoreType.DMA((2,)),
        ],
    )
    return kb(h4, src2, dst2, zeros32)


R = 2000
G = N // R


def _k0_body(x_ref, w_ref, b_ref, eap_ref, h_ref, h4_ref, eaf_ref):
    h = jnp.dot(x_ref[...], w_ref[...], precision=_P,
                preferred_element_type=jnp.float32) + b_ref[...]
    h_ref[...] = h
    for gg in range(CH):
        h4_ref[gg] = h[:, gg * CW:(gg + 1) * CW]
    eaf_ref[...] = eap_ref[0] + eap_ref[1]


def _tc_in(x, w, b, eap):
    return pl.pallas_call(
        _k0_body,
        grid=(G,),
        in_specs=[
            pl.BlockSpec((R, x.shape[1]), lambda i: (i, 0)),
            pl.BlockSpec(w.shape, lambda i: (0, 0)),
            pl.BlockSpec((1, HID), lambda i: (0, 0)),
            pl.BlockSpec((2, R, 16), lambda i: (0, i, 0)),
        ],
        out_specs=[
            pl.BlockSpec((R, HID), lambda i: (i, 0)),
            pl.BlockSpec((CH, R, CW), lambda i: (0, i, 0)),
            pl.BlockSpec((R, 16), lambda i: (i, 0)),
        ],
        out_shape=[
            jax.ShapeDtypeStruct((N, HID), jnp.float32),
            jax.ShapeDtypeStruct((CH, N, CW), jnp.float32),
            jax.ShapeDtypeStruct((N, 16), jnp.float32),
        ],
    )(x, w, b, eap)


def _k1_body(h_ref, a4_ref, ea_ref, wx_ref, w1_ref, b1_ref, eps_ref,
             u_ref, st_ref):
    agg = jnp.concatenate([a4_ref[gg] for gg in range(CH)], axis=-1)
    pre = ((1.0 + eps_ref[0, 0]) * h_ref[...] + agg
           + jnp.dot(ea_ref[...], wx_ref[...], precision=_P,
                     preferred_element_type=jnp.float32))
    u = jnp.dot(pre, w1_ref[...], precision=_P,
                preferred_element_type=jnp.float32) + b1_ref[...]
    u_ref[...] = u

    @pl.when(pl.program_id(0) == 0)
    def _():
        st_ref[...] = jnp.zeros_like(st_ref)

    rid = lax.broadcasted_iota(jnp.int32, st_ref.shape, 0)
    st_ref[...] += jnp.where(rid == 0, jnp.sum(u, axis=0, keepdims=True),
                             jnp.where(rid == 1,
                                       jnp.sum(u * u, axis=0, keepdims=True),
                                       0.0))


def _tc_k1(h4, agg4, eaf, wx, w1, b1, eps):
    d2 = w1.shape[1]
    return pl.pallas_call(
        _k1_body,
        grid=(G,),
        in_specs=[
            pl.BlockSpec((R, HID), lambda i: (i, 0)),
            pl.BlockSpec((CH, R, CW), lambda i: (0, i, 0)),
            pl.BlockSpec((R, 16), lambda i: (i, 0)),
            pl.BlockSpec((16, HID), lambda i: (0, 0)),
            pl.BlockSpec((HID, d2), lambda i: (0, 0)),
            pl.BlockSpec((1, d2), lambda i: (0, 0)),
            pl.BlockSpec((1, 1), lambda i: (0, 0)),
        ],
        out_specs=[
            pl.BlockSpec((R, d2), lambda i: (i, 0)),
            pl.BlockSpec((8, d2), lambda i: (0, 0)),
        ],
        out_shape=[
            jax.ShapeDtypeStruct((N, d2), jnp.float32),
            jax.ShapeDtypeStruct((8, d2), jnp.float32),
        ],
    )(h4, agg4, eaf, wx, w1, b1, eps)


def _k2_body(u_ref, st_ref, g_ref, bt_ref, w_ref, b_ref, v_ref, st2_ref):
    m = st_ref[0:1, :] / N
    var = jnp.maximum(st_ref[1:2, :] / N - m * m, 0.0)
    a = g_ref[...] * lax.rsqrt(var + 1e-5)
    cb = bt_ref[...] - m * a
    un = jnp.maximum(u_ref[...] * a + cb, 0.0)
    v = jnp.dot(un, w_ref[...], precision=_P,
                preferred_element_type=jnp.float32) + b_ref[...]
    v_ref[...] = v

    @pl.when(pl.program_id(0) == 0)
    def _():
        st2_ref[...] = jnp.zeros_like(st2_ref)

    rid = lax.broadcasted_iota(jnp.int32, st2_ref.shape, 0)
    st2_ref[...] += jnp.where(rid == 0, jnp.sum(v, axis=0, keepdims=True),
                              jnp.where(rid == 1,
                                        jnp.sum(v * v, axis=0, keepdims=True),
                                        0.0))


def _tc_k2(u, st1, g1, bt1, w2, b2):
    d1 = u.shape[1]
    d2 = w2.shape[1]
    return pl.pallas_call(
        _k2_body,
        grid=(G,),
        in_specs=[
            pl.BlockSpec((R, d1), lambda i: (i, 0)),
            pl.BlockSpec((8, d1), lambda i: (0, 0)),
            pl.BlockSpec((1, d1), lambda i: (0, 0)),
            pl.BlockSpec((1, d1), lambda i: (0, 0)),
            pl.BlockSpec((d1, d2), lambda i: (0, 0)),
            pl.BlockSpec((1, d2), lambda i: (0, 0)),
        ],
        out_specs=[
            pl.BlockSpec((R, d2), lambda i: (i, 0)),
            pl.BlockSpec((8, d2), lambda i: (0, 0)),
        ],
        out_shape=[
            jax.ShapeDtypeStruct((N, d2), jnp.float32),
            jax.ShapeDtypeStruct((8, d2), jnp.float32),
        ],
    )(u, st1, g1, bt1, w2, b2)


def _k3_body(v_ref, st_ref, g_ref, bt_ref, h_ref, lg_ref, lb_ref,
             h_out, h4_out):
    m = st_ref[0:1, :] / N
    var = jnp.maximum(st_ref[1:2, :] / N - m * m, 0.0)
    a = g_ref[...] * lax.rsqrt(var + 1e-5)
    cb = bt_ref[...] - m * a
    t = jnp.maximum(v_ref[...] * a + cb, 0.0) + h_ref[...]
    lm = jnp.mean(t, axis=-1, keepdims=True)
    lv = jnp.mean((t - lm) * (t - lm), axis=-1, keepdims=True)
    hn = (t - lm) * lax.rsqrt(lv + 1e-5) * lg_ref[...] + lb_ref[...]
    h_out[...] = hn
    for gg in range(CH):
        h4_out[gg] = hn[:, gg * CW:(gg + 1) * CW]


def _tc_k3(v, st2, g2, bt2, h, lg, lb):
    return pl.pallas_call(
        _k3_body,
        grid=(G,),
        in_specs=[
            pl.BlockSpec((R, HID), lambda i: (i, 0)),
            pl.BlockSpec((8, HID), lambda i: (0, 0)),
            pl.BlockSpec((1, HID), lambda i: (0, 0)),
            pl.BlockSpec((1, HID), lambda i: (0, 0)),
            pl.BlockSpec((R, HID), lambda i: (i, 0)),
            pl.BlockSpec((1, HID), lambda i: (0, 0)),
            pl.BlockSpec((1, HID), lambda i: (0, 0)),
        ],
        out_specs=[
            pl.BlockSpec((R, HID), lambda i: (i, 0)),
            pl.BlockSpec((CH, R, CW), lambda i: (0, i, 0)),
        ],
        out_shape=[
            jax.ShapeDtypeStruct((N, HID), jnp.float32),
            jax.ShapeDtypeStruct((CH, N, CW), jnp.float32),
        ],
    )(v, st2, g2, bt2, h, lg, lb)


def _k4_body(h0_ref, h1_ref, h2_ref, h3_ref, w10, w11, w12, w13, b1_ref,
             w2_ref, b2_ref, batch_ref, ne_ref, ge_ref, psum, pcnt):
    hr = [h0_ref, h1_ref, h2_ref, h3_ref]
    wr = [w10, w11, w12, w13]
    acc = b1_ref[...]
    for ii in range(4):
        acc += jnp.dot(hr[ii][...], wr[ii][...], precision=_P,
                       preferred_element_type=jnp.float32)
    acc = jnp.maximum(acc, 0.0)
    ne = jnp.dot(acc, w2_ref[...], precision=_P,
                 preferred_element_type=jnp.float32) + b2_ref[...]
    ne_ref[...] = ne

    @pl.when(pl.program_id(0) == 0)
    def _():
        psum[...] = jnp.zeros_like(psum)
        pcnt[...] = jnp.zeros_like(pcnt)

    bt = batch_ref[0, 0, :]
    onehot = (bt[:, None] == lax.broadcasted_iota(jnp.int32, (R, NG), 1))
    onehot = onehot.astype(jnp.float32)
    dn = (((0,), (0,)), ((), ()))
    psum[...] += lax.dot_general(onehot, ne, dn, precision=_P,
                                 preferred_element_type=jnp.float32)
    pcnt[...] += lax.dot_general(onehot, jnp.ones((R, HID), jnp.float32), dn,
                                 precision=_P,
                                 preferred_element_type=jnp.float32)

    @pl.when(pl.program_id(0) == G - 1)
    def _():
        ge_ref[...] = psum[...] / jnp.maximum(pcnt[...], 1.0)


def _tc_k4(hs, w1s, b1, w2, b2, batch3):
    return pl.pallas_call(
        _k4_body,
        grid=(G,),
        in_specs=(
            [pl.BlockSpec((R, HID), lambda i: (i, 0)) for _ in range(4)]
            + [pl.BlockSpec((HID, HID), lambda i: (0, 0)) for _ in range(4)]
            + [
                pl.BlockSpec((1, HID), lambda i: (0, 0)),
                pl.BlockSpec((HID, HID), lambda i: (0, 0)),
                pl.BlockSpec((1, HID), lambda i: (0, 0)),
                pl.BlockSpec((1, 1, R), lambda i: (i, 0, 0)),
            ]
        ),
        out_specs=[
            pl.BlockSpec((R, HID), lambda i: (i, 0)),
            pl.BlockSpec((NG, HID), lambda i: (0, 0)),
        ],
        out_shape=[
            jax.ShapeDtypeStruct((N, HID), jnp.float32),
            jax.ShapeDtypeStruct((NG, HID), jnp.float32),
        ],
        scratch_shapes=[
            pltpu.VMEM((NG, HID), jnp.float32),
            pltpu.VMEM((NG, HID), jnp.float32),
        ],
    )(*hs, *w1s, b1, w2, b2, batch3)


def kernel(x, edge_attr, edge_index, batch, params):
    pad = EP - E
    src2 = jnp.concatenate(
        [edge_index[0], jnp.zeros((pad,), jnp.int32)]).reshape(TB, B)
    dst2 = jnp.concatenate(
        [edge_index[1], jnp.full((pad,), N, jnp.int32)]).reshape(TB, B)
    ea_pad3 = jnp.concatenate(
        [jnp.concatenate([edge_attr, jnp.ones((E, 1), jnp.float32),
                          jnp.zeros((E, 12), jnp.float32)], axis=1),
         jnp.zeros((pad, 16), jnp.float32)], axis=0).reshape(TB, B, 16)
    zeros16 = jnp.zeros((NP, 16), jnp.float32)
    zeros32 = jnp.zeros((NP, CW), jnp.float32)
    batch3 = batch.reshape(G, 1, R)

    eap = _sc_ea(ea_pad3, dst2, zeros16)

    h, h4, eaf = _tc_in(x, params['in_W'], params['in_b'].reshape(1, HID),
                        eap)
    hs = [h]
    for lp in params['layers']:
        wx = jnp.concatenate(
            [lp['We'], lp['be'].reshape(1, HID),
             jnp.zeros((12, HID), jnp.float32)], axis=0)
        agg4 = _sc_agg(h4, src2, dst2, zeros32)
        u, st1 = _tc_k1(h, agg4, eaf, wx, lp['W1'],
                        lp['b1'].reshape(1, -1), lp['eps'].reshape(1, 1))
        v, st2 = _tc_k2(u, st1, lp['g1'].reshape(1, -1),
                        lp['bt1'].reshape(1, -1), lp['W2'],
                        lp['b2'].reshape(1, -1))
        h, h4 = _tc_k3(v, st2, lp['g2'].reshape(1, -1),
                       lp['bt2'].reshape(1, -1), h,
                       lp['ln_g'].reshape(1, -1), lp['ln_b'].reshape(1, -1))
        hs.append(h)

    w1s = [params['out_W1'][i * HID:(i + 1) * HID] for i in range(4)]
    node_emb, graph_emb = _tc_k4(
        hs, w1s, params['out_b1'].reshape(1, HID), params['out_W2'],
        params['out_b2'].reshape(1, HID), batch3)
    return node_emb, graph_emb


# speedup vs baseline: 1.1326x; 1.0744x over previous
import functools

import jax
import jax.numpy as jnp
from jax import lax
from jax.experimental import pallas as pl
from jax.experimental.pallas import tpu as pltpu
from jax.experimental.pallas import tpu_sc as plsc

N = 50000
E = 800000
HID = 128
NG = 64

NCORES = 2
NSUB = 16
B = 128
TB = 6400
EP = TB * B
KSLOT = 5
DDELAY = 4
NP = 50048
NPT = NP // NSUB
CH = 4
CW = HID // CH

_P = jax.lax.Precision.DEFAULT


def _ea_body(ea_hbm, dst_hbm, zeros_hbm, out_hbm, dstv, rows, shared, semr, sems):
    c = lax.axis_index("c")
    s = lax.axis_index("s")
    nb = TB // (NCORES * NSUB)
    b0 = c * (TB // NCORES) + s * nb

    pltpu.sync_copy(zeros_hbm.at[pl.ds(s * NPT, NPT)],
                    shared.at[pl.ds(s * NPT, NPT)])
    plsc.subcore_barrier()

    pltpu.sync_copy(dst_hbm.at[pl.ds(b0, nb)], dstv)

    mceil = (nb + DDELAY + KSLOT - 1) // KSLOT

    def step(m, _):
        for k in range(KSLOT):
            i = m * KSLOT + k
            q = (k - DDELAY) % KSLOT

            @pl.when(i < nb)
            def _():
                @pl.when(i >= KSLOT)
                def _():
                    pltpu.make_async_copy(
                        rows.at[k], shared.at[dstv.at[i - KSLOT]],
                        sems.at[k]).wait()
                pltpu.async_copy(ea_hbm.at[b0 + i], rows.at[k], semr.at[k])

            j = i - DDELAY

            @pl.when(jnp.logical_and(j >= 0, j < nb))
            def _():
                pltpu.make_async_copy(ea_hbm.at[b0 + j], rows.at[q],
                                      semr.at[q]).wait()
                pltpu.async_copy(rows.at[q], shared.at[dstv.at[j]],
                                 sems.at[q], add=True)
        return 0

    lax.fori_loop(0, mceil, step, 0)

    for k in range(KSLOT):
        jj = nb - 1 - ((nb - 1 - k) % KSLOT)
        pltpu.make_async_copy(rows.at[jj % KSLOT], shared.at[dstv.at[jj]],
                              sems.at[jj % KSLOT]).wait()

    plsc.subcore_barrier()
    pltpu.sync_copy(shared.at[pl.ds(s * NPT, NPT)],
                    out_hbm.at[c, pl.ds(s * NPT, NPT)])


def _sc_ea(ea_pad3, dst2, zeros16):
    mesh = plsc.VectorSubcoreMesh(core_axis_name="c", subcore_axis_name="s")
    kb = pl.kernel(
        _ea_body,
        out_type=jax.ShapeDtypeStruct((NCORES, NP, 16), jnp.float32),
        mesh=mesh,
        compiler_params=pltpu.CompilerParams(use_tc_tiling_on_sc=False),
        scratch_types=[
            pltpu.VMEM((TB // (NCORES * NSUB), B), jnp.int32),
            pltpu.VMEM((KSLOT, B, 16), jnp.float32),
            pltpu.VMEM_SHARED((NP, 16), jnp.float32),
            pltpu.SemaphoreType.DMA((KSLOT,)),
            pltpu.SemaphoreType.DMA((KSLOT,)),
        ],
    )
    return kb(ea_pad3, dst2, zeros16)


SB = 10
NSUP = (TB // NSUB) // SB


def _agg_body(h_hbm, src_hbm, dst_hbm, zeros_hbm, out_hbm,
              srcv, dstv, rows, shared, semr, sems, semi):
    c = lax.axis_index("c")
    s = lax.axis_index("s")
    nb = TB // NSUB
    b0 = s * nb

    def idx_issue(msup):
        sl = lax.rem(msup, 2)
        pltpu.async_copy(src_hbm.at[pl.ds(b0 + msup * SB, SB)],
                         srcv.at[sl], semi.at[sl])
        pltpu.async_copy(dst_hbm.at[pl.ds(b0 + msup * SB, SB)],
                         dstv.at[sl], semi.at[sl])

    def idx_wait(msup):
        sl = lax.rem(msup, 2)
        pltpu.make_async_copy(src_hbm.at[pl.ds(b0 + msup * SB, SB)],
                              srcv.at[sl], semi.at[sl]).wait()
        pltpu.make_async_copy(dst_hbm.at[pl.ds(b0 + msup * SB, SB)],
                              dstv.at[sl], semi.at[sl]).wait()

    def srow(msup, k2):
        return srcv.at[lax.rem(msup + (k2 // SB), 2), k2 % SB]

    def drow(msup, k2):
        return dstv.at[lax.rem(msup + (k2 // SB), 2), k2 % SB]

    for cc in range(CH // NCORES):
        g = c * (CH // NCORES) + cc

        pltpu.sync_copy(zeros_hbm.at[pl.ds(s * NPT, NPT)],
                        shared.at[pl.ds(s * NPT, NPT)])
        plsc.subcore_barrier()
        idx_issue(0)

        def step(msup, _):
            idx_wait(msup)
            for k2 in range(SB):
                i = msup * SB + k2
                k = k2 % KSLOT
                q = (k2 - DDELAY) % KSLOT

                @pl.when(i >= KSLOT)
                def _():
                    pltpu.make_async_copy(
                        rows.at[k], shared.at[drow(msup, k2 - KSLOT)],
                        sems.at[k]).wait()

                pltpu.async_copy(h_hbm.at[g].at[srow(msup, k2)],
                                 rows.at[k], semr.at[k])

                j = i - DDELAY

                @pl.when(j >= 0)
                def _():
                    pltpu.make_async_copy(
                        h_hbm.at[g].at[srow(msup, k2 - DDELAY)],
                        rows.at[q], semr.at[q]).wait()
                    pltpu.async_copy(rows.at[q],
                                     shared.at[drow(msup, k2 - DDELAY)],
                                     sems.at[q], add=True)

                if k2 == KSLOT:
                    @pl.when(msup + 1 < NSUP)
                    def _():
                        idx_issue(msup + 1)
            return 0

        lax.fori_loop(0, NSUP, step, 0)

        last = NSUP - 1
        for k2 in range(SB, SB + DDELAY):
            q = (k2 - DDELAY) % KSLOT
            jx = last * SB + k2 - DDELAY
            pltpu.make_async_copy(h_hbm.at[g].at[srow(last, k2 - DDELAY)],
                                  rows.at[q], semr.at[q]).wait()
            pltpu.async_copy(rows.at[q], shared.at[drow(last, k2 - DDELAY)],
                             sems.at[q], add=True)
        for k2 in range(SB - KSLOT, SB):
            k = k2 % KSLOT
            jx = last * SB + k2
            pltpu.make_async_copy(rows.at[k], shared.at[drow(last, k2)],
                                  sems.at[k]).wait()

        plsc.subcore_barrier()
        pltpu.sync_copy(shared.at[pl.ds(s * NPT, NPT)],
                        out_hbm.at[g, pl.ds(s * NPT, NPT)])
        plsc.subcore_barrier()


def _sc_agg(h4, src2, dst2, zeros32):
    mesh = plsc.VectorSubcoreMesh(core_axis_name="c", subcore_axis_name="s")
    kb = pl.kernel(
        _agg_body,
        out_type=jax.ShapeDtypeStruct((CH, NP, CW), jnp.float32),
        mesh=mesh,
        compiler_params=pltpu.CompilerParams(use_tc_tiling_on_sc=False),
        scratch_types=[
            pltpu.VMEM((2, SB, B), jnp.int32),
            pltpu.VMEM((2, SB, B), jnp.int32),
            pltpu.VMEM((KSLOT, B, CW), jnp.float32),
            pltpu.VMEM_SHARED((NP, CW), jnp.float32),
            pltpu.SemaphoreType.DMA((KSLOT,)),
            pltpu.SemaphoreType.DMA((KSLOT,)),
            pltpu.SemaphoreType.DMA((2,)),
        ],
    )
    return kb(h4, src2, dst2, zeros32)


R = 2000
G = N // R


def _k0_body(x_ref, w_ref, b_ref, h_ref, h4_ref):
    h = jnp.dot(x_ref[...], w_ref[...], precision=_P,
                preferred_element_type=jnp.float32) + b_ref[...]
    h_ref[...] = h
    for gg in range(CH):
        h4_ref[gg] = h[:, gg * CW:(gg + 1) * CW]


def _tc_in(x, w, b):
    return pl.pallas_call(
        _k0_body,
        grid=(G,),
        in_specs=[
            pl.BlockSpec((R, x.shape[1]), lambda i: (i, 0)),
            pl.BlockSpec(w.shape, lambda i: (0, 0)),
            pl.BlockSpec((1, HID), lambda i: (0, 0)),
        ],
        out_specs=[
            pl.BlockSpec((R, HID), lambda i: (i, 0)),
            pl.BlockSpec((CH, R, CW), lambda i: (0, i, 0)),
        ],
        out_shape=[
            jax.ShapeDtypeStruct((N, HID), jnp.float32),
            jax.ShapeDtypeStruct((CH, N, CW), jnp.float32),
        ],
    )(x, w, b)


def _k1_body(h_ref, a4_ref, ea_ref, wx_ref, w1_ref, b1_ref, eps_ref,
             u_ref, st_ref):
    agg = jnp.concatenate([a4_ref[gg] for gg in range(CH)], axis=-1)
    eaf = ea_ref[0] + ea_ref[1]
    pre = ((1.0 + eps_ref[0, 0]) * h_ref[...] + agg
           + jnp.dot(eaf, wx_ref[...], precision=_P,
                     preferred_element_type=jnp.float32))
    u = jnp.dot(pre, w1_ref[...], precision=_P,
                preferred_element_type=jnp.float32) + b1_ref[...]
    u_ref[...] = u

    @pl.when(pl.program_id(0) == 0)
    def _():
        st_ref[...] = jnp.zeros_like(st_ref)

    rid = lax.broadcasted_iota(jnp.int32, st_ref.shape, 0)
    st_ref[...] += jnp.where(rid == 0, jnp.sum(u, axis=0, keepdims=True),
                             jnp.where(rid == 1,
                                       jnp.sum(u * u, axis=0, keepdims=True),
                                       0.0))


def _tc_k1(h4, agg4, eaf, wx, w1, b1, eps):
    d2 = w1.shape[1]
    return pl.pallas_call(
        _k1_body,
        grid=(G,),
        in_specs=[
            pl.BlockSpec((R, HID), lambda i: (i, 0)),
            pl.BlockSpec((CH, R, CW), lambda i: (0, i, 0)),
            pl.BlockSpec((2, R, 16), lambda i: (0, i, 0)),
            pl.BlockSpec((16, HID), lambda i: (0, 0)),
            pl.BlockSpec((HID, d2), lambda i: (0, 0)),
            pl.BlockSpec((1, d2), lambda i: (0, 0)),
            pl.BlockSpec((1, 1), lambda i: (0, 0)),
        ],
        out_specs=[
            pl.BlockSpec((R, d2), lambda i: (i, 0)),
            pl.BlockSpec((8, d2), lambda i: (0, 0)),
        ],
        out_shape=[
            jax.ShapeDtypeStruct((N, d2), jnp.float32),
            jax.ShapeDtypeStruct((8, d2), jnp.float32),
        ],
    )(h4, agg4, eaf, wx, w1, b1, eps)


def _k2_body(u_ref, st_ref, g_ref, bt_ref, w_ref, b_ref, v_ref, st2_ref):
    m = st_ref[0:1, :] / N
    var = jnp.maximum(st_ref[1:2, :] / N - m * m, 0.0)
    a = g_ref[...] * lax.rsqrt(var + 1e-5)
    cb = bt_ref[...] - m * a
    un = jnp.maximum(u_ref[...] * a + cb, 0.0)
    v = jnp.dot(un, w_ref[...], precision=_P,
                preferred_element_type=jnp.float32) + b_ref[...]
    v_ref[...] = v

    @pl.when(pl.program_id(0) == 0)
    def _():
        st2_ref[...] = jnp.zeros_like(st2_ref)

    rid = lax.broadcasted_iota(jnp.int32, st2_ref.shape, 0)
    st2_ref[...] += jnp.where(rid == 0, jnp.sum(v, axis=0, keepdims=True),
                              jnp.where(rid == 1,
                                        jnp.sum(v * v, axis=0, keepdims=True),
                                        0.0))


def _tc_k2(u, st1, g1, bt1, w2, b2):
    d1 = u.shape[1]
    d2 = w2.shape[1]
    return pl.pallas_call(
        _k2_body,
        grid=(G,),
        in_specs=[
            pl.BlockSpec((R, d1), lambda i: (i, 0)),
            pl.BlockSpec((8, d1), lambda i: (0, 0)),
            pl.BlockSpec((1, d1), lambda i: (0, 0)),
            pl.BlockSpec((1, d1), lambda i: (0, 0)),
            pl.BlockSpec((d1, d2), lambda i: (0, 0)),
            pl.BlockSpec((1, d2), lambda i: (0, 0)),
        ],
        out_specs=[
            pl.BlockSpec((R, d2), lambda i: (i, 0)),
            pl.BlockSpec((8, d2), lambda i: (0, 0)),
        ],
        out_shape=[
            jax.ShapeDtypeStruct((N, d2), jnp.float32),
            jax.ShapeDtypeStruct((8, d2), jnp.float32),
        ],
    )(u, st1, g1, bt1, w2, b2)


def _k3_body(v_ref, st_ref, g_ref, bt_ref, h_ref, lg_ref, lb_ref,
             h_out, h4_out):
    m = st_ref[0:1, :] / N
    var = jnp.maximum(st_ref[1:2, :] / N - m * m, 0.0)
    a = g_ref[...] * lax.rsqrt(var + 1e-5)
    cb = bt_ref[...] - m * a
    t = jnp.maximum(v_ref[...] * a + cb, 0.0) + h_ref[...]
    lm = jnp.mean(t, axis=-1, keepdims=True)
    lv = jnp.mean((t - lm) * (t - lm), axis=-1, keepdims=True)
    hn = (t - lm) * lax.rsqrt(lv + 1e-5) * lg_ref[...] + lb_ref[...]
    h_out[...] = hn
    for gg in range(CH):
        h4_out[gg] = hn[:, gg * CW:(gg + 1) * CW]


def _tc_k3(v, st2, g2, bt2, h, lg, lb):
    return pl.pallas_call(
        _k3_body,
        grid=(G,),
        in_specs=[
            pl.BlockSpec((R, HID), lambda i: (i, 0)),
            pl.BlockSpec((8, HID), lambda i: (0, 0)),
            pl.BlockSpec((1, HID), lambda i: (0, 0)),
            pl.BlockSpec((1, HID), lambda i: (0, 0)),
            pl.BlockSpec((R, HID), lambda i: (i, 0)),
            pl.BlockSpec((1, HID), lambda i: (0, 0)),
            pl.BlockSpec((1, HID), lambda i: (0, 0)),
        ],
        out_specs=[
            pl.BlockSpec((R, HID), lambda i: (i, 0)),
            pl.BlockSpec((CH, R, CW), lambda i: (0, i, 0)),
        ],
        out_shape=[
            jax.ShapeDtypeStruct((N, HID), jnp.float32),
            jax.ShapeDtypeStruct((CH, N, CW), jnp.float32),
        ],
    )(v, st2, g2, bt2, h, lg, lb)


def _k4_body(h0_ref, h1_ref, h2_ref, h3_ref, w10, w11, w12, w13, b1_ref,
             w2_ref, b2_ref, batch_ref, ne_ref, ge_ref, psum, pcnt):
    hr = [h0_ref, h1_ref, h2_ref, h3_ref]
    wr = [w10, w11, w12, w13]
    acc = b1_ref[...]
    for ii in range(4):
        acc += jnp.dot(hr[ii][...], wr[ii][...], precision=_P,
                       preferred_element_type=jnp.float32)
    acc = jnp.maximum(acc, 0.0)
    ne = jnp.dot(acc, w2_ref[...], precision=_P,
                 preferred_element_type=jnp.float32) + b2_ref[...]
    ne_ref[...] = ne

    @pl.when(pl.program_id(0) == 0)
    def _():
        psum[...] = jnp.zeros_like(psum)
        pcnt[...] = jnp.zeros_like(pcnt)

    bt = batch_ref[0, 0, :]
    onehot = (bt[:, None] == lax.broadcasted_iota(jnp.int32, (R, NG), 1))
    onehot = onehot.astype(jnp.float32)
    dn = (((0,), (0,)), ((), ()))
    psum[...] += lax.dot_general(onehot, ne, dn, precision=_P,
                                 preferred_element_type=jnp.float32)
    pcnt[...] += lax.dot_general(onehot, jnp.ones((R, HID), jnp.float32), dn,
                                 precision=_P,
                                 preferred_element_type=jnp.float32)

    @pl.when(pl.program_id(0) == G - 1)
    def _():
        ge_ref[...] = psum[...] / jnp.maximum(pcnt[...], 1.0)


def _tc_k4(hs, w1s, b1, w2, b2, batch3):
    return pl.pallas_call(
        _k4_body,
        grid=(G,),
        in_specs=(
            [pl.BlockSpec((R, HID), lambda i: (i, 0)) for _ in range(4)]
            + [pl.BlockSpec((HID, HID), lambda i: (0, 0)) for _ in range(4)]
            + [
                pl.BlockSpec((1, HID), lambda i: (0, 0)),
                pl.BlockSpec((HID, HID), lambda i: (0, 0)),
                pl.BlockSpec((1, HID), lambda i: (0, 0)),
                pl.BlockSpec((1, 1, R), lambda i: (i, 0, 0)),
            ]
        ),
        out_specs=[
            pl.BlockSpec((R, HID), lambda i: (i, 0)),
            pl.BlockSpec((NG, HID), lambda i: (0, 0)),
        ],
        out_shape=[
            jax.ShapeDtypeStruct((N, HID), jnp.float32),
            jax.ShapeDtypeStruct((NG, HID), jnp.float32),
        ],
        scratch_shapes=[
            pltpu.VMEM((NG, HID), jnp.float32),
            pltpu.VMEM((NG, HID), jnp.float32),
        ],
    )(*hs, *w1s, b1, w2, b2, batch3)


def kernel(x, edge_attr, edge_index, batch, params):
    pad = EP - E
    src2 = jnp.concatenate(
        [edge_index[0], jnp.zeros((pad,), jnp.int32)]).reshape(TB, B)
    dst2 = jnp.concatenate(
        [edge_index[1], jnp.full((pad,), N, jnp.int32)]).reshape(TB, B)
    ea_pad3 = jnp.concatenate(
        [jnp.concatenate([edge_attr, jnp.ones((E, 1), jnp.float32),
                          jnp.zeros((E, 12), jnp.float32)], axis=1),
         jnp.zeros((pad, 16), jnp.float32)], axis=0).reshape(TB, B, 16)
    zeros16 = jnp.zeros((NP, 16), jnp.float32)
    zeros32 = jnp.zeros((NP, CW), jnp.float32)
    batch3 = batch.reshape(G, 1, R)

    eap = _sc_ea(ea_pad3, dst2, zeros16)

    h, h4 = _tc_in(x, params['in_W'], params['in_b'].reshape(1, HID))
    hs = [h]
    for lp in params['layers']:
        wx = jnp.concatenate(
            [lp['We'], lp['be'].reshape(1, HID),
             jnp.zeros((12, HID), jnp.float32)], axis=0)
        agg4 = _sc_agg(h4, src2, dst2, zeros32)
        u, st1 = _tc_k1(h, agg4, eap, wx, lp['W1'],
                        lp['b1'].reshape(1, -1), lp['eps'].reshape(1, 1))
        v, st2 = _tc_k2(u, st1, lp['g1'].reshape(1, -1),
                        lp['bt1'].reshape(1, -1), lp['W2'],
                        lp['b2'].reshape(1, -1))
        h, h4 = _tc_k3(v, st2, lp['g2'].reshape(1, -1),
                       lp['bt2'].reshape(1, -1), h,
                       lp['ln_g'].reshape(1, -1), lp['ln_b'].reshape(1, -1))
        hs.append(h)

    w1s = [params['out_W1'][i * HID:(i + 1) * HID] for i in range(4)]
    node_emb, graph_emb = _tc_k4(
        hs, w1s, params['out_b1'].reshape(1, HID), params['out_W2'],
        params['out_b2'].reshape(1, HID), batch3)
    return node_emb, graph_emb


# speedup vs baseline: 1.1488x; 1.0143x over previous
import functools

import jax
import jax.numpy as jnp
from jax import lax
from jax.experimental import pallas as pl
from jax.experimental.pallas import tpu as pltpu
from jax.experimental.pallas import tpu_sc as plsc

N = 50000
E = 800000
HID = 128
NG = 64

NCORES = 2
NSUB = 16
B = 128
TB = 6400
EP = TB * B
KSLOT = 5
DDELAY = 4
NP = 50048
NPT = NP // NSUB
CH = 4
CW = HID // CH

_P = jax.lax.Precision.DEFAULT


def _ea_body(ea_hbm, dst_hbm, zeros_hbm, out_hbm, dstv, rows, shared, semr, sems):
    c = lax.axis_index("c")
    s = lax.axis_index("s")
    nb = TB // (NCORES * NSUB)
    b0 = c * (TB // NCORES) + s * nb

    pltpu.sync_copy(zeros_hbm.at[pl.ds(s * NPT, NPT)],
                    shared.at[pl.ds(s * NPT, NPT)])
    plsc.subcore_barrier()

    pltpu.sync_copy(dst_hbm.at[pl.ds(b0, nb)], dstv)

    mceil = (nb + DDELAY + KSLOT - 1) // KSLOT

    def step(m, _):
        for k in range(KSLOT):
            i = m * KSLOT + k
            q = (k - DDELAY) % KSLOT

            @pl.when(i < nb)
            def _():
                @pl.when(i >= KSLOT)
                def _():
                    pltpu.make_async_copy(
                        rows.at[k], shared.at[dstv.at[i - KSLOT]],
                        sems.at[k]).wait()
                pltpu.async_copy(ea_hbm.at[b0 + i], rows.at[k], semr.at[k])

            j = i - DDELAY

            @pl.when(jnp.logical_and(j >= 0, j < nb))
            def _():
                pltpu.make_async_copy(ea_hbm.at[b0 + j], rows.at[q],
                                      semr.at[q]).wait()
                pltpu.async_copy(rows.at[q], shared.at[dstv.at[j]],
                                 sems.at[q], add=True)
        return 0

    lax.fori_loop(0, mceil, step, 0)

    for k in range(KSLOT):
        jj = nb - 1 - ((nb - 1 - k) % KSLOT)
        pltpu.make_async_copy(rows.at[jj % KSLOT], shared.at[dstv.at[jj]],
                              sems.at[jj % KSLOT]).wait()

    plsc.subcore_barrier()
    pltpu.sync_copy(shared.at[pl.ds(s * NPT, NPT)],
                    out_hbm.at[c, pl.ds(s * NPT, NPT)])


def _sc_ea(ea_pad3, dst2, zeros16):
    mesh = plsc.VectorSubcoreMesh(core_axis_name="c", subcore_axis_name="s")
    kb = pl.kernel(
        _ea_body,
        out_type=jax.ShapeDtypeStruct((NCORES, NP, 16), jnp.float32),
        mesh=mesh,
        compiler_params=pltpu.CompilerParams(use_tc_tiling_on_sc=False),
        scratch_types=[
            pltpu.VMEM((TB // (NCORES * NSUB), B), jnp.int32),
            pltpu.VMEM((KSLOT, B, 16), jnp.float32),
            pltpu.VMEM_SHARED((NP, 16), jnp.float32),
            pltpu.SemaphoreType.DMA((KSLOT,)),
            pltpu.SemaphoreType.DMA((KSLOT,)),
        ],
    )
    return kb(ea_pad3, dst2, zeros16)


SB = 10
NSUP = (TB // NSUB) // SB


def _agg_body(h_hbm, src_hbm, dst_hbm, zeros_hbm, out_hbm,
              srcv, dstv, rows, shared, semr, sems, semi):
    c = lax.axis_index("c")
    s = lax.axis_index("s")
    nb = TB // NSUB
    b0 = s * nb

    def idx_issue(msup):
        sl = lax.rem(msup, 2)
        pltpu.async_copy(src_hbm.at[pl.ds(b0 + msup * SB, SB)],
                         srcv.at[sl], semi.at[sl])
        pltpu.async_copy(dst_hbm.at[pl.ds(b0 + msup * SB, SB)],
                         dstv.at[sl], semi.at[sl])

    def idx_wait(msup):
        sl = lax.rem(msup, 2)
        pltpu.make_async_copy(src_hbm.at[pl.ds(b0 + msup * SB, SB)],
                              srcv.at[sl], semi.at[sl]).wait()
        pltpu.make_async_copy(dst_hbm.at[pl.ds(b0 + msup * SB, SB)],
                              dstv.at[sl], semi.at[sl]).wait()

    def srow(msup, k2):
        return srcv.at[lax.rem(msup + (k2 // SB), 2), k2 % SB]

    def drow(msup, k2):
        return dstv.at[lax.rem(msup + (k2 // SB), 2), k2 % SB]

    for cc in range(CH // NCORES):
        g = c * (CH // NCORES) + cc

        pltpu.sync_copy(zeros_hbm.at[pl.ds(s * NPT, NPT)],
                        shared.at[pl.ds(s * NPT, NPT)])
        plsc.subcore_barrier()
        idx_issue(0)

        def step(msup, _):
            idx_wait(msup)
            for k2 in range(SB):
                i = msup * SB + k2
                k = k2 % KSLOT
                q = (k2 - DDELAY) % KSLOT

                @pl.when(i >= KSLOT)
                def _():
                    pltpu.make_async_copy(
                        rows.at[k], shared.at[drow(msup, k2 - KSLOT)],
                        sems.at[k]).wait()

                pltpu.async_copy(h_hbm.at[g].at[srow(msup, k2)],
                                 rows.at[k], semr.at[k])

                j = i - DDELAY

                @pl.when(j >= 0)
                def _():
                    pltpu.make_async_copy(
                        h_hbm.at[g].at[srow(msup, k2 - DDELAY)],
                        rows.at[q], semr.at[q]).wait()
                    pltpu.async_copy(rows.at[q],
                                     shared.at[drow(msup, k2 - DDELAY)],
                                     sems.at[q], add=True)

                if k2 == KSLOT:
                    @pl.when(msup + 1 < NSUP)
                    def _():
                        idx_issue(msup + 1)
            return 0

        lax.fori_loop(0, NSUP, step, 0)

        last = NSUP - 1
        for k2 in range(SB, SB + DDELAY):
            q = (k2 - DDELAY) % KSLOT
            jx = last * SB + k2 - DDELAY
            pltpu.make_async_copy(h_hbm.at[g].at[srow(last, k2 - DDELAY)],
                                  rows.at[q], semr.at[q]).wait()
            pltpu.async_copy(rows.at[q], shared.at[drow(last, k2 - DDELAY)],
                             sems.at[q], add=True)
        for k2 in range(SB - KSLOT, SB):
            k = k2 % KSLOT
            jx = last * SB + k2
            pltpu.make_async_copy(rows.at[k], shared.at[drow(last, k2)],
                                  sems.at[k]).wait()

        plsc.subcore_barrier()
        pltpu.sync_copy(shared.at[pl.ds(s * NPT, NPT)],
                        out_hbm.at[g, pl.ds(s * NPT, NPT)])
        plsc.subcore_barrier()


def _sc_agg(h4, src2, dst2, zeros32):
    mesh = plsc.VectorSubcoreMesh(core_axis_name="c", subcore_axis_name="s")
    kb = pl.kernel(
        _agg_body,
        out_type=jax.ShapeDtypeStruct((CH, NP, CW), jnp.float32),
        mesh=mesh,
        compiler_params=pltpu.CompilerParams(use_tc_tiling_on_sc=False),
        scratch_types=[
            pltpu.VMEM((2, SB, B), jnp.int32),
            pltpu.VMEM((2, SB, B), jnp.int32),
            pltpu.VMEM((KSLOT, B, CW), jnp.float32),
            pltpu.VMEM_SHARED((NP, CW), jnp.float32),
            pltpu.SemaphoreType.DMA((KSLOT,)),
            pltpu.SemaphoreType.DMA((KSLOT,)),
            pltpu.SemaphoreType.DMA((2,)),
        ],
    )
    return kb(h4, src2, dst2, zeros32)


R = 2000
G = N // R


def _k0_body(x_ref, w_ref, b_ref, h_ref, h4_ref):
    h = jnp.dot(x_ref[...], w_ref[...], precision=_P,
                preferred_element_type=jnp.float32) + b_ref[...]
    h_ref[...] = h
    for gg in range(CH):
        h4_ref[gg] = h[:, gg * CW:(gg + 1) * CW]


def _tc_in(x, w, b):
    return pl.pallas_call(
        _k0_body,
        grid=(G,),
        in_specs=[
            pl.BlockSpec((R, x.shape[1]), lambda i: (i, 0)),
            pl.BlockSpec(w.shape, lambda i: (0, 0)),
            pl.BlockSpec((1, HID), lambda i: (0, 0)),
        ],
        out_specs=[
            pl.BlockSpec((R, HID), lambda i: (i, 0)),
            pl.BlockSpec((CH, R, CW), lambda i: (0, i, 0)),
        ],
        out_shape=[
            jax.ShapeDtypeStruct((N, HID), jnp.float32),
            jax.ShapeDtypeStruct((CH, N, CW), jnp.float32),
        ],
    )(x, w, b)


def _k1_body(h_ref, a4_ref, ea_ref, wx_ref, w1_ref, b1_ref, eps_ref,
             u_ref, st_ref):
    agg = jnp.concatenate([a4_ref[gg] for gg in range(CH)], axis=-1)
    eaf = ea_ref[0] + ea_ref[1]
    pre = ((1.0 + eps_ref[0, 0]) * h_ref[...] + agg
           + jnp.dot(eaf, wx_ref[...], precision=_P,
                     preferred_element_type=jnp.float32))
    u = jnp.dot(pre, w1_ref[...], precision=_P,
                preferred_element_type=jnp.float32) + b1_ref[...]
    u_ref[...] = u.astype(jnp.bfloat16)

    @pl.when(pl.program_id(0) == 0)
    def _():
        st_ref[...] = jnp.zeros_like(st_ref)

    rid = lax.broadcasted_iota(jnp.int32, st_ref.shape, 0)
    st_ref[...] += jnp.where(rid == 0, jnp.sum(u, axis=0, keepdims=True),
                             jnp.where(rid == 1,
                                       jnp.sum(u * u, axis=0, keepdims=True),
                                       0.0))


def _tc_k1(h4, agg4, eaf, wx, w1, b1, eps):
    d2 = w1.shape[1]
    return pl.pallas_call(
        _k1_body,
        grid=(G,),
        in_specs=[
            pl.BlockSpec((R, HID), lambda i: (i, 0)),
            pl.BlockSpec((CH, R, CW), lambda i: (0, i, 0)),
            pl.BlockSpec((2, R, 16), lambda i: (0, i, 0)),
            pl.BlockSpec((16, HID), lambda i: (0, 0)),
            pl.BlockSpec((HID, d2), lambda i: (0, 0)),
            pl.BlockSpec((1, d2), lambda i: (0, 0)),
            pl.BlockSpec((1, 1), lambda i: (0, 0)),
        ],
        out_specs=[
            pl.BlockSpec((R, d2), lambda i: (i, 0)),
            pl.BlockSpec((8, d2), lambda i: (0, 0)),
        ],
        out_shape=[
            jax.ShapeDtypeStruct((N, d2), jnp.bfloat16),
            jax.ShapeDtypeStruct((8, d2), jnp.float32),
        ],
    )(h4, agg4, eaf, wx, w1, b1, eps)


def _k2_body(u_ref, st_ref, g_ref, bt_ref, w_ref, b_ref, v_ref, st2_ref):
    m = st_ref[0:1, :] / N
    var = jnp.maximum(st_ref[1:2, :] / N - m * m, 0.0)
    a = g_ref[...] * lax.rsqrt(var + 1e-5)
    cb = bt_ref[...] - m * a
    un = jnp.maximum(u_ref[...].astype(jnp.float32) * a + cb, 0.0)
    v = jnp.dot(un, w_ref[...], precision=_P,
                preferred_element_type=jnp.float32) + b_ref[...]
    v_ref[...] = v

    @pl.when(pl.program_id(0) == 0)
    def _():
        st2_ref[...] = jnp.zeros_like(st2_ref)

    rid = lax.broadcasted_iota(jnp.int32, st2_ref.shape, 0)
    st2_ref[...] += jnp.where(rid == 0, jnp.sum(v, axis=0, keepdims=True),
                              jnp.where(rid == 1,
                                        jnp.sum(v * v, axis=0, keepdims=True),
                                        0.0))


def _tc_k2(u, st1, g1, bt1, w2, b2):
    d1 = u.shape[1]
    d2 = w2.shape[1]
    return pl.pallas_call(
        _k2_body,
        grid=(G,),
        in_specs=[
            pl.BlockSpec((R, d1), lambda i: (i, 0)),
            pl.BlockSpec((8, d1), lambda i: (0, 0)),
            pl.BlockSpec((1, d1), lambda i: (0, 0)),
            pl.BlockSpec((1, d1), lambda i: (0, 0)),
            pl.BlockSpec((d1, d2), lambda i: (0, 0)),
            pl.BlockSpec((1, d2), lambda i: (0, 0)),
        ],
        out_specs=[
            pl.BlockSpec((R, d2), lambda i: (i, 0)),
            pl.BlockSpec((8, d2), lambda i: (0, 0)),
        ],
        out_shape=[
            jax.ShapeDtypeStruct((N, d2), jnp.float32),
            jax.ShapeDtypeStruct((8, d2), jnp.float32),
        ],
    )(u, st1, g1, bt1, w2, b2)


def _k3_body(v_ref, st_ref, g_ref, bt_ref, h_ref, lg_ref, lb_ref,
             h_out, h4_out):
    m = st_ref[0:1, :] / N
    var = jnp.maximum(st_ref[1:2, :] / N - m * m, 0.0)
    a = g_ref[...] * lax.rsqrt(var + 1e-5)
    cb = bt_ref[...] - m * a
    t = jnp.maximum(v_ref[...] * a + cb, 0.0) + h_ref[...]
    lm = jnp.mean(t, axis=-1, keepdims=True)
    lv = jnp.mean((t - lm) * (t - lm), axis=-1, keepdims=True)
    hn = (t - lm) * lax.rsqrt(lv + 1e-5) * lg_ref[...] + lb_ref[...]
    h_out[...] = hn
    for gg in range(CH):
        h4_out[gg] = hn[:, gg * CW:(gg + 1) * CW]


def _tc_k3(v, st2, g2, bt2, h, lg, lb):
    return pl.pallas_call(
        _k3_body,
        grid=(G,),
        in_specs=[
            pl.BlockSpec((R, HID), lambda i: (i, 0)),
            pl.BlockSpec((8, HID), lambda i: (0, 0)),
            pl.BlockSpec((1, HID), lambda i: (0, 0)),
            pl.BlockSpec((1, HID), lambda i: (0, 0)),
            pl.BlockSpec((R, HID), lambda i: (i, 0)),
            pl.BlockSpec((1, HID), lambda i: (0, 0)),
            pl.BlockSpec((1, HID), lambda i: (0, 0)),
        ],
        out_specs=[
            pl.BlockSpec((R, HID), lambda i: (i, 0)),
            pl.BlockSpec((CH, R, CW), lambda i: (0, i, 0)),
        ],
        out_shape=[
            jax.ShapeDtypeStruct((N, HID), jnp.float32),
            jax.ShapeDtypeStruct((CH, N, CW), jnp.float32),
        ],
    )(v, st2, g2, bt2, h, lg, lb)


def _k4_body(h0_ref, h1_ref, h2_ref, h3_ref, w10, w11, w12, w13, b1_ref,
             w2_ref, b2_ref, batch_ref, ne_ref, ge_ref, psum, pcnt):
    hr = [h0_ref, h1_ref, h2_ref, h3_ref]
    wr = [w10, w11, w12, w13]
    acc = b1_ref[...]
    for ii in range(4):
        acc += jnp.dot(hr[ii][...], wr[ii][...], precision=_P,
                       preferred_element_type=jnp.float32)
    acc = jnp.maximum(acc, 0.0)
    ne = jnp.dot(acc, w2_ref[...], precision=_P,
                 preferred_element_type=jnp.float32) + b2_ref[...]
    ne_ref[...] = ne

    @pl.when(pl.program_id(0) == 0)
    def _():
        psum[...] = jnp.zeros_like(psum)
        pcnt[...] = jnp.zeros_like(pcnt)

    bt = batch_ref[0, 0, :]
    onehot = (bt[:, None] == lax.broadcasted_iota(jnp.int32, (R, NG), 1))
    onehot = onehot.astype(jnp.float32)
    dn = (((0,), (0,)), ((), ()))
    psum[...] += lax.dot_general(onehot, ne, dn, precision=_P,
                                 preferred_element_type=jnp.float32)
    pcnt[...] += lax.dot_general(onehot, jnp.ones((R, HID), jnp.float32), dn,
                                 precision=_P,
                                 preferred_element_type=jnp.float32)

    @pl.when(pl.program_id(0) == G - 1)
    def _():
        ge_ref[...] = psum[...] / jnp.maximum(pcnt[...], 1.0)


def _tc_k4(hs, w1s, b1, w2, b2, batch3):
    return pl.pallas_call(
        _k4_body,
        grid=(G,),
        in_specs=(
            [pl.BlockSpec((R, HID), lambda i: (i, 0)) for _ in range(4)]
            + [pl.BlockSpec((HID, HID), lambda i: (0, 0)) for _ in range(4)]
            + [
                pl.BlockSpec((1, HID), lambda i: (0, 0)),
                pl.BlockSpec((HID, HID), lambda i: (0, 0)),
                pl.BlockSpec((1, HID), lambda i: (0, 0)),
                pl.BlockSpec((1, 1, R), lambda i: (i, 0, 0)),
            ]
        ),
        out_specs=[
            pl.BlockSpec((R, HID), lambda i: (i, 0)),
            pl.BlockSpec((NG, HID), lambda i: (0, 0)),
        ],
        out_shape=[
            jax.ShapeDtypeStruct((N, HID), jnp.float32),
            jax.ShapeDtypeStruct((NG, HID), jnp.float32),
        ],
        scratch_shapes=[
            pltpu.VMEM((NG, HID), jnp.float32),
            pltpu.VMEM((NG, HID), jnp.float32),
        ],
    )(*hs, *w1s, b1, w2, b2, batch3)


def kernel(x, edge_attr, edge_index, batch, params):
    pad = EP - E
    src2 = jnp.concatenate(
        [edge_index[0], jnp.zeros((pad,), jnp.int32)]).reshape(TB, B)
    dst2 = jnp.concatenate(
        [edge_index[1], jnp.full((pad,), N, jnp.int32)]).reshape(TB, B)
    ea_pad3 = jnp.concatenate(
        [jnp.concatenate([edge_attr, jnp.ones((E, 1), jnp.float32),
                          jnp.zeros((E, 12), jnp.float32)], axis=1),
         jnp.zeros((pad, 16), jnp.float32)], axis=0).reshape(TB, B, 16)
    zeros16 = jnp.zeros((NP, 16), jnp.float32)
    zeros32 = jnp.zeros((NP, CW), jnp.float32)
    batch3 = batch.reshape(G, 1, R)

    eap = _sc_ea(ea_pad3, dst2, zeros16)

    h, h4 = _tc_in(x, params['in_W'], params['in_b'].reshape(1, HID))
    hs = [h]
    for lp in params['layers']:
        wx = jnp.concatenate(
            [lp['We'], lp['be'].reshape(1, HID),
             jnp.zeros((12, HID), jnp.float32)], axis=0)
        agg4 = _sc_agg(h4, src2, dst2, zeros32)
        u, st1 = _tc_k1(h, agg4, eap, wx, lp['W1'],
                        lp['b1'].reshape(1, -1), lp['eps'].reshape(1, 1))
        v, st2 = _tc_k2(u, st1, lp['g1'].reshape(1, -1),
                        lp['bt1'].reshape(1, -1), lp['W2'],
                        lp['b2'].reshape(1, -1))
        h, h4 = _tc_k3(v, st2, lp['g2'].reshape(1, -1),
                       lp['bt2'].reshape(1, -1), h,
                       lp['ln_g'].reshape(1, -1), lp['ln_b'].reshape(1, -1))
        hs.append(h)

    w1s = [params['out_W1'][i * HID:(i + 1) * HID] for i in range(4)]
    node_emb, graph_emb = _tc_k4(
        hs, w1s, params['out_b1'].reshape(1, HID), params['out_W2'],
        params['out_b2'].reshape(1, HID), batch3)
    return node_emb, graph_emb
